# Initial kernel scaffold; baseline (speedup 1.0000x reference)
import jax
import jax.numpy as jnp
from jax.experimental import pallas as pl


def kernel(x, edge_index, W_hp, b_hp, W_lp, b_lp, W_i, b_i, wh, bh, wl, bl, wi, bi):
    raise NotImplementedError("write your pallas kernel here")


# speedup vs baseline: 12.5541x; 12.5541x over previous
import functools

import jax
import jax.numpy as jnp
from jax import lax
from jax.experimental import pallas as pl
from jax.experimental.pallas import tpu as pltpu
from jax.experimental.pallas import tpu_sc as plsc

N = 10000
E = 320000
D = 128
NC = 2
NS = 16
K = 80
RPT = 624
TAIL = N - NS * RPT
EPW_A = E // (NC * NS)
EPT_C = E // NS
CH_A = EPW_A // K
CH_C = EPT_C // K
BN = 1000


def _sc_mesh():
    return plsc.VectorSubcoreMesh(core_axis_name="c", subcore_axis_name="s")


def _m8(v):
    return pl.multiple_of(v, 8)


def _deg_body(dst1d, zer, one, out, acc, obuf, didx, sem):
    del sem
    c = lax.axis_index("c")
    s = lax.axis_index("s")
    w = s * NC + c
    b0 = _m8(s * RPT)
    pltpu.sync_copy(zer, acc.at[pl.ds(b0, RPT), :])

    @pl.when(s == NS - 1)
    def _():
        pltpu.sync_copy(zer.at[pl.ds(0, TAIL), :],
                        acc.at[pl.ds(_m8(NS * RPT), TAIL), :])

    pltpu.sync_copy(one, obuf)
    plsc.subcore_barrier()

    def chunk(i, carry):
        e0 = _m8(w * EPW_A + i * K)
        pltpu.sync_copy(dst1d.at[pl.ds(e0, K)], didx)
        pltpu.sync_copy(obuf, acc.at[didx], add=True)
        return carry

    lax.fori_loop(0, CH_A, chunk, 0)
    plsc.subcore_barrier()
    pltpu.sync_copy(acc.at[pl.ds(b0, RPT), :],
                    out.at[pl.ds(_m8(c * N + b0), RPT), :])

    @pl.when(s == NS - 1)
    def _():
        pltpu.sync_copy(acc.at[pl.ds(_m8(NS * RPT), TAIL), :],
                        out.at[pl.ds(_m8(c * N + NS * RPT), TAIL), :])


_deg_kernel = functools.partial(
    pl.kernel,
    out_type=jax.ShapeDtypeStruct((NC * N, D), jnp.float32),
    mesh=_sc_mesh(),
    scratch_types=[
        pltpu.VMEM_SHARED((N, D), jnp.float32),
        pltpu.VMEM((K, D), jnp.float32),
        pltpu.VMEM((K,), jnp.int32),
        pltpu.SemaphoreType.DMA,
    ],
)(_deg_body)


def _edge_body(hp, src1d, dst1d, out, acc, rows, sidx, didx, sem):
    c = lax.axis_index("c")
    s = lax.axis_index("s")
    off = c * N
    b0 = _m8(s * RPT)
    pltpu.sync_copy(hp.at[pl.ds(_m8(off + b0), RPT), :],
                    acc.at[pl.ds(b0, RPT), :])

    @pl.when(s == NS - 1)
    def _():
        pltpu.sync_copy(hp.at[pl.ds(_m8(off + NS * RPT), TAIL), :],
                        acc.at[pl.ds(_m8(NS * RPT), TAIL), :])

    plsc.subcore_barrier()

    def chunk(i, carry):
        e0 = _m8(c * E + s * EPT_C + i * K)
        pltpu.sync_copy(src1d.at[pl.ds(e0, K)], sidx)
        pltpu.sync_copy(dst1d.at[pl.ds(_m8(s * EPT_C + i * K), K)], didx)
        pltpu.async_copy(hp.at[sidx], rows, sem).wait()
        pltpu.sync_copy(rows, acc.at[didx], add=True)
        return carry

    lax.fori_loop(0, CH_C, chunk, 0)
    plsc.subcore_barrier()
    pltpu.sync_copy(acc.at[pl.ds(b0, RPT), :],
                    out.at[pl.ds(_m8(off + b0), RPT), :])

    @pl.when(s == NS - 1)
    def _():
        pltpu.sync_copy(acc.at[pl.ds(_m8(NS * RPT), TAIL), :],
                        out.at[pl.ds(_m8(off + NS * RPT), TAIL), :])


_edge_kernel = functools.partial(
    pl.kernel,
    out_type=jax.ShapeDtypeStruct((NC * N, D), jnp.float32),
    mesh=_sc_mesh(),
    scratch_types=[
        pltpu.VMEM_SHARED((N, D), jnp.float32),
        pltpu.VMEM((K, D), jnp.float32),
        pltpu.VMEM((K,), jnp.int32),
        pltpu.VMEM((K,), jnp.int32),
        pltpu.SemaphoreType.DMA,
    ],
)(_edge_body)


def _dense_body(x_ref, wlp_ref, whp_ref, wi_ref, blp_ref, bhp_ref, bi_ref,
                degp_ref, hp_ref, hhp_ref, hi_ref):
    x = x_ref[...]
    d = degp_ref[...]
    deg = d[0, :, 0:1] + d[1, :, 0:1] + 1.0
    dinv = lax.rsqrt(deg)
    hlp = jnp.dot(x, wlp_ref[...], preferred_element_type=jnp.float32) + blp_ref[...]
    hhp = jnp.dot(x, whp_ref[...], preferred_element_type=jnp.float32) + bhp_ref[...]
    hi = jnp.dot(x, wi_ref[...], preferred_element_type=jnp.float32) + bi_ref[...]
    hp_ref[0, :, :] = dinv * hlp
    hp_ref[1, :, :] = dinv * hhp
    hhp_ref[...] = hhp
    hi_ref[...] = jnp.maximum(hi, 0.0)


def _dense_stage(x, wlp, whp, wi, blp, bhp, bi, degp):
    full = pl.BlockSpec((D, D), lambda i: (0, 0))
    bias = pl.BlockSpec((1, D), lambda i: (0, 0))
    row = pl.BlockSpec((BN, D), lambda i: (i, 0))
    return pl.pallas_call(
        _dense_body,
        grid=(N // BN,),
        in_specs=[row, full, full, full, bias, bias, bias,
                  pl.BlockSpec((NC, BN, D), lambda i: (0, i, 0))],
        out_specs=[pl.BlockSpec((NC, BN, D), lambda i: (0, i, 0)), row, row],
        out_shape=[
            jax.ShapeDtypeStruct((NC, N, D), jnp.float32),
            jax.ShapeDtypeStruct((N, D), jnp.float32),
            jax.ShapeDtypeStruct((N, D), jnp.float32),
        ],
    )(x, wlp, whp, wi, blp, bhp, bi, degp)


def _epi_body(s_ref, hhp_ref, hi_ref, degp_ref, gw_ref, gb_ref, o_ref):
    d = degp_ref[...]
    deg = d[0, :, 0:1] + d[1, :, 0:1] + 1.0
    dinv = lax.rsqrt(deg)
    s = s_ref[...]
    h_lp = jnp.maximum(dinv * s[0], 0.0)
    h_hp = jnp.maximum(hhp_ref[...] - dinv * s[1], 0.0)
    h_i = hi_ref[...]
    gw = gw_ref[...]
    gb = gb_ref[...]

    def gate(h, k):
        z = jnp.sum(h * gw[k:k + 1, :], axis=1, keepdims=True) + gb[k:k + 1, 0:1]
        return 1.0 / (1.0 + jnp.exp(-z))

    out = gate(h_hp, 0) * h_hp + gate(h_lp, 1) * h_lp + gate(h_i, 2) * h_i
    m = jnp.max(out, axis=1, keepdims=True)
    lse = jnp.log(jnp.sum(jnp.exp(out - m), axis=1, keepdims=True)) + m
    o_ref[...] = out - lse


def _epi_stage(s3, hhp, hi, degp, gw, gb):
    row = pl.BlockSpec((BN, D), lambda i: (i, 0))
    return pl.pallas_call(
        _epi_body,
        grid=(N // BN,),
        in_specs=[pl.BlockSpec((NC, BN, D), lambda i: (0, i, 0)), row, row,
                  pl.BlockSpec((NC, BN, D), lambda i: (0, i, 0)),
                  pl.BlockSpec((3, D), lambda i: (0, 0)),
                  pl.BlockSpec((3, D), lambda i: (0, 0))],
        out_specs=row,
        out_shape=jax.ShapeDtypeStruct((N, D), jnp.float32),
    )(s3, hhp, hi, degp, gw, gb)


def kernel(x, edge_index, W_hp, b_hp, W_lp, b_lp, W_i, b_i, wh, bh, wl, bl, wi, bi):
    src = edge_index[0].astype(jnp.int32)
    dst = edge_index[1].astype(jnp.int32)
    zer = jnp.zeros((RPT, D), jnp.float32)
    one = jnp.ones((K, D), jnp.float32)

    degp = _deg_kernel(dst, zer, one).reshape(NC, N, D)

    hp, hhp, hi = _dense_stage(
        x, W_lp, W_hp, W_i,
        b_lp.reshape(1, D), b_hp.reshape(1, D), b_i.reshape(1, D), degp)

    src2 = jnp.concatenate([src, src + N])
    s2 = _edge_kernel(hp.reshape(NC * N, D), src2, dst)

    gw = jnp.concatenate([wh, wl, wi], axis=1).T
    gb = jnp.broadcast_to(jnp.concatenate([bh, bl, bi])[:, None], (3, D))
    return _epi_stage(s2.reshape(NC, N, D), hhp, hi, degp, gw, gb)


# speedup vs baseline: 25.6174x; 2.0406x over previous
import functools

import jax
import jax.numpy as jnp
from jax import lax
from jax.experimental import pallas as pl
from jax.experimental.pallas import tpu as pltpu
from jax.experimental.pallas import tpu_sc as plsc

N = 10000
E = 320000
D = 128
NC = 2
NS = 16
K = 80
RPT = 624
TAIL = N - NS * RPT
EPW_A = E // (NC * NS)
EPT_C = E // NS
CH_A = EPW_A // K
CH_C = EPT_C // K
NB_A = 25
NB_C = 4
NG_C = CH_C // NB_C
BN = 1000


def _sc_mesh():
    return plsc.VectorSubcoreMesh(core_axis_name="c", subcore_axis_name="s")


def _m8(v):
    return pl.multiple_of(v, 8)


def _deg_body(dst1d, zer, one, out, acc, obuf, didx, semi, sems):
    c = lax.axis_index("c")
    s = lax.axis_index("s")
    w = s * NC + c
    b0 = _m8(s * RPT)
    pltpu.sync_copy(zer, acc.at[pl.ds(b0, RPT), :])

    @pl.when(s == NS - 1)
    def _():
        pltpu.sync_copy(zer.at[pl.ds(0, TAIL), :],
                        acc.at[pl.ds(_m8(NS * RPT), TAIL), :])

    pltpu.sync_copy(one, obuf)
    plsc.subcore_barrier()

    def group(g, carry):
        loads = []
        for j in range(NB_A):
            e0 = _m8(w * EPW_A + (g * NB_A + j) * K)
            loads.append(pltpu.async_copy(dst1d.at[pl.ds(e0, K)], didx.at[j], semi))
        scats = []
        for j in range(NB_A):
            loads[j].wait()
            scats.append(pltpu.async_copy(obuf, acc.at[didx.at[j]], sems, add=True))
        for cp in scats:
            cp.wait()
        return carry

    lax.fori_loop(0, CH_A // NB_A, group, 0)
    plsc.subcore_barrier()
    pltpu.sync_copy(acc.at[pl.ds(b0, RPT), :],
                    out.at[pl.ds(_m8(c * N + b0), RPT), :])

    @pl.when(s == NS - 1)
    def _():
        pltpu.sync_copy(acc.at[pl.ds(_m8(NS * RPT), TAIL), :],
                        out.at[pl.ds(_m8(c * N + NS * RPT), TAIL), :])


_deg_kernel = functools.partial(
    pl.kernel,
    out_type=jax.ShapeDtypeStruct((NC * N, D), jnp.float32),
    mesh=_sc_mesh(),
    scratch_types=[
        pltpu.VMEM_SHARED((N, D), jnp.float32),
        pltpu.VMEM((K, D), jnp.float32),
        pltpu.VMEM((NB_A, K), jnp.int32),
        pltpu.SemaphoreType.DMA,
        pltpu.SemaphoreType.DMA,
    ],
)(_deg_body)


def _edge_body(hp, src1d, dst1d, out, acc, rows, sidx, didx, semi, semg, sems):
    c = lax.axis_index("c")
    s = lax.axis_index("s")
    off = c * N
    b0 = _m8(s * RPT)
    pltpu.sync_copy(hp.at[pl.ds(_m8(off + b0), RPT), :],
                    acc.at[pl.ds(b0, RPT), :])

    @pl.when(s == NS - 1)
    def _():
        pltpu.sync_copy(hp.at[pl.ds(_m8(off + NS * RPT), TAIL), :],
                        acc.at[pl.ds(_m8(NS * RPT), TAIL), :])

    plsc.subcore_barrier()

    def chunks(base, nb):
        lsrc = [pltpu.async_copy(
            src1d.at[pl.ds(_m8(c * E + (base + j) * K), K)], sidx.at[j], semi)
            for j in range(nb)]
        ldst = [pltpu.async_copy(
            dst1d.at[pl.ds(_m8((base + j) * K), K)], didx.at[j], semi)
            for j in range(nb)]
        gats = []
        for j in range(nb):
            lsrc[j].wait()
            gats.append(pltpu.async_copy(hp.at[sidx.at[j]], rows.at[j], semg))
        scats = []
        for j in range(nb):
            gats[j].wait()
            ldst[j].wait()
            scats.append(pltpu.async_copy(rows.at[j], acc.at[didx.at[j]],
                                          sems, add=True))
        for cp in scats:
            cp.wait()

    def group(g, carry):
        chunks(s * CH_C + g * NB_C, NB_C)
        return carry

    lax.fori_loop(0, NG_C, group, 0)
    chunks(s * CH_C + NG_C * NB_C, CH_C - NG_C * NB_C)
    plsc.subcore_barrier()
    pltpu.sync_copy(acc.at[pl.ds(b0, RPT), :],
                    out.at[pl.ds(_m8(off + b0), RPT), :])

    @pl.when(s == NS - 1)
    def _():
        pltpu.sync_copy(acc.at[pl.ds(_m8(NS * RPT), TAIL), :],
                        out.at[pl.ds(_m8(off + NS * RPT), TAIL), :])


_edge_kernel = functools.partial(
    pl.kernel,
    out_type=jax.ShapeDtypeStruct((NC * N, D), jnp.float32),
    mesh=_sc_mesh(),
    scratch_types=[
        pltpu.VMEM_SHARED((N, D), jnp.float32),
        pltpu.VMEM((NB_C, K, D), jnp.float32),
        pltpu.VMEM((NB_C, K), jnp.int32),
        pltpu.VMEM((NB_C, K), jnp.int32),
        pltpu.SemaphoreType.DMA,
        pltpu.SemaphoreType.DMA,
        pltpu.SemaphoreType.DMA,
    ],
)(_edge_body)


def _dense_body(x_ref, wlp_ref, whp_ref, wi_ref, blp_ref, bhp_ref, bi_ref,
                degp_ref, hp_ref, hhp_ref, hi_ref):
    x = x_ref[...]
    d = degp_ref[...]
    deg = d[0, :, 0:1] + d[1, :, 0:1] + 1.0
    dinv = lax.rsqrt(deg)
    hlp = jnp.dot(x, wlp_ref[...], preferred_element_type=jnp.float32) + blp_ref[...]
    hhp = jnp.dot(x, whp_ref[...], preferred_element_type=jnp.float32) + bhp_ref[...]
    hi = jnp.dot(x, wi_ref[...], preferred_element_type=jnp.float32) + bi_ref[...]
    hp_ref[0, :, :] = dinv * hlp
    hp_ref[1, :, :] = dinv * hhp
    hhp_ref[...] = hhp
    hi_ref[...] = jnp.maximum(hi, 0.0)


def _dense_stage(x, wlp, whp, wi, blp, bhp, bi, degp):
    full = pl.BlockSpec((D, D), lambda i: (0, 0))
    bias = pl.BlockSpec((1, D), lambda i: (0, 0))
    row = pl.BlockSpec((BN, D), lambda i: (i, 0))
    return pl.pallas_call(
        _dense_body,
        grid=(N // BN,),
        in_specs=[row, full, full, full, bias, bias, bias,
                  pl.BlockSpec((NC, BN, D), lambda i: (0, i, 0))],
        out_specs=[pl.BlockSpec((NC, BN, D), lambda i: (0, i, 0)), row, row],
        out_shape=[
            jax.ShapeDtypeStruct((NC, N, D), jnp.float32),
            jax.ShapeDtypeStruct((N, D), jnp.float32),
            jax.ShapeDtypeStruct((N, D), jnp.float32),
        ],
    )(x, wlp, whp, wi, blp, bhp, bi, degp)


def _epi_body(s_ref, hhp_ref, hi_ref, degp_ref, gw_ref, gb_ref, o_ref):
    d = degp_ref[...]
    deg = d[0, :, 0:1] + d[1, :, 0:1] + 1.0
    dinv = lax.rsqrt(deg)
    s = s_ref[...]
    h_lp = jnp.maximum(dinv * s[0], 0.0)
    h_hp = jnp.maximum(hhp_ref[...] - dinv * s[1], 0.0)
    h_i = hi_ref[...]
    gw = gw_ref[...]
    gb = gb_ref[...]

    def gate(h, k):
        z = jnp.sum(h * gw[k:k + 1, :], axis=1, keepdims=True) + gb[k:k + 1, 0:1]
        return 1.0 / (1.0 + jnp.exp(-z))

    out = gate(h_hp, 0) * h_hp + gate(h_lp, 1) * h_lp + gate(h_i, 2) * h_i
    m = jnp.max(out, axis=1, keepdims=True)
    lse = jnp.log(jnp.sum(jnp.exp(out - m), axis=1, keepdims=True)) + m
    o_ref[...] = out - lse


def _epi_stage(s3, hhp, hi, degp, gw, gb):
    row = pl.BlockSpec((BN, D), lambda i: (i, 0))
    return pl.pallas_call(
        _epi_body,
        grid=(N // BN,),
        in_specs=[pl.BlockSpec((NC, BN, D), lambda i: (0, i, 0)), row, row,
                  pl.BlockSpec((NC, BN, D), lambda i: (0, i, 0)),
                  pl.BlockSpec((3, D), lambda i: (0, 0)),
                  pl.BlockSpec((3, D), lambda i: (0, 0))],
        out_specs=row,
        out_shape=jax.ShapeDtypeStruct((N, D), jnp.float32),
    )(s3, hhp, hi, degp, gw, gb)


def kernel(x, edge_index, W_hp, b_hp, W_lp, b_lp, W_i, b_i, wh, bh, wl, bl, wi, bi):
    src = edge_index[0].astype(jnp.int32)
    dst = edge_index[1].astype(jnp.int32)
    zer = jnp.zeros((RPT, D), jnp.float32)
    one = jnp.ones((K, D), jnp.float32)

    degp = _deg_kernel(dst, zer, one).reshape(NC, N, D)

    hp, hhp, hi = _dense_stage(
        x, W_lp, W_hp, W_i,
        b_lp.reshape(1, D), b_hp.reshape(1, D), b_i.reshape(1, D), degp)

    src2 = jnp.concatenate([src, src + N])
    s2 = _edge_kernel(hp.reshape(NC * N, D), src2, dst)

    gw = jnp.concatenate([wh, wl, wi], axis=1).T
    gb = jnp.broadcast_to(jnp.concatenate([bh, bl, bi])[:, None], (3, D))
    return _epi_stage(s2.reshape(NC, N, D), hhp, hi, degp, gw, gb)


# speedup vs baseline: 25.6600x; 1.0017x over previous
import functools

import jax
import jax.numpy as jnp
from jax import lax
from jax.experimental import pallas as pl
from jax.experimental.pallas import tpu as pltpu
from jax.experimental.pallas import tpu_sc as plsc

N = 10000
E = 320000
D = 128
NC = 2
NS = 16
K = 80
RPT = 624
TAIL = N - NS * RPT
EPW_A = E // (NC * NS)
EPT_C = E // NS
CH_A = EPW_A // K
CH_C = EPT_C // K
NB_A = 25
NB_C = 4
NG_C = CH_C // NB_C
BN = 1000


def _sc_mesh():
    return plsc.VectorSubcoreMesh(core_axis_name="c", subcore_axis_name="s")


def _m8(v):
    return pl.multiple_of(v, 8)


def _deg_body(dst1d, zer, one, out, acc, obuf, didx, semi, sems):
    c = lax.axis_index("c")
    s = lax.axis_index("s")
    w = s * NC + c
    b0 = _m8(s * RPT)
    pltpu.sync_copy(zer, acc.at[pl.ds(b0, RPT), :])

    @pl.when(s == NS - 1)
    def _():
        pltpu.sync_copy(zer.at[pl.ds(0, TAIL), :],
                        acc.at[pl.ds(_m8(NS * RPT), TAIL), :])

    pltpu.sync_copy(one, obuf)
    plsc.subcore_barrier()

    def group(g, carry):
        loads = []
        for j in range(NB_A):
            e0 = _m8(w * EPW_A + (g * NB_A + j) * K)
            loads.append(pltpu.async_copy(dst1d.at[pl.ds(e0, K)], didx.at[j], semi))
        scats = []
        for j in range(NB_A):
            loads[j].wait()
            scats.append(pltpu.async_copy(obuf, acc.at[didx.at[j]], sems, add=True))
        for cp in scats:
            cp.wait()
        return carry

    lax.fori_loop(0, CH_A // NB_A, group, 0)
    plsc.subcore_barrier()
    pltpu.sync_copy(acc.at[pl.ds(b0, RPT), :],
                    out.at[pl.ds(_m8(c * N + b0), RPT), :])

    @pl.when(s == NS - 1)
    def _():
        pltpu.sync_copy(acc.at[pl.ds(_m8(NS * RPT), TAIL), :],
                        out.at[pl.ds(_m8(c * N + NS * RPT), TAIL), :])


_deg_kernel = functools.partial(
    pl.kernel,
    out_type=jax.ShapeDtypeStruct((NC * N, D), jnp.float32),
    mesh=_sc_mesh(),
    scratch_types=[
        pltpu.VMEM_SHARED((N, D), jnp.float32),
        pltpu.VMEM((K, D), jnp.float32),
        pltpu.VMEM((NB_A, K), jnp.int32),
        pltpu.SemaphoreType.DMA,
        pltpu.SemaphoreType.DMA,
    ],
)(_deg_body)


def _edge_body(hp, src1d, dst1d, out, acc, rows, sidx, didx, semi, semg, sems):
    c = lax.axis_index("c")
    s = lax.axis_index("s")
    off = c * N
    b0 = _m8(s * RPT)
    pltpu.sync_copy(hp.at[pl.ds(_m8(off + b0), RPT), :],
                    acc.at[pl.ds(b0, RPT), :])

    @pl.when(s == NS - 1)
    def _():
        pltpu.sync_copy(hp.at[pl.ds(_m8(off + NS * RPT), TAIL), :],
                        acc.at[pl.ds(_m8(NS * RPT), TAIL), :])

    plsc.subcore_barrier()

    def chunks(base, nb):
        lsrc = [pltpu.async_copy(
            src1d.at[pl.ds(_m8(c * E + (base + j) * K), K)], sidx.at[j], semi)
            for j in range(nb)]
        ldst = [pltpu.async_copy(
            dst1d.at[pl.ds(_m8((base + j) * K), K)], didx.at[j], semi)
            for j in range(nb)]
        gats = []
        for j in range(nb):
            lsrc[j].wait()
            gats.append(pltpu.async_copy(hp.at[sidx.at[j]], rows.at[j], semg))
        scats = []
        for j in range(nb):
            gats[j].wait()
            ldst[j].wait()
            scats.append(pltpu.async_copy(rows.at[j], acc.at[didx.at[j]],
                                          sems, add=True))
        for cp in scats:
            cp.wait()

    def group(g, carry):
        chunks(s * CH_C + g * NB_C, NB_C)
        return carry

    lax.fori_loop(0, NG_C, group, 0)
    chunks(s * CH_C + NG_C * NB_C, CH_C - NG_C * NB_C)
    plsc.subcore_barrier()
    pltpu.sync_copy(acc.at[pl.ds(b0, RPT), :],
                    out.at[pl.ds(_m8(off + b0), RPT), :])

    @pl.when(s == NS - 1)
    def _():
        pltpu.sync_copy(acc.at[pl.ds(_m8(NS * RPT), TAIL), :],
                        out.at[pl.ds(_m8(off + NS * RPT), TAIL), :])


_edge_kernel = functools.partial(
    pl.kernel,
    out_type=jax.ShapeDtypeStruct((NC * N, D), jnp.float32),
    mesh=_sc_mesh(),
    scratch_types=[
        pltpu.VMEM_SHARED((N, D), jnp.float32),
        pltpu.VMEM((NB_C, K, D), jnp.float32),
        pltpu.VMEM((NB_C, K), jnp.int32),
        pltpu.VMEM((NB_C, K), jnp.int32),
        pltpu.SemaphoreType.DMA,
        pltpu.SemaphoreType.DMA,
        pltpu.SemaphoreType.DMA,
    ],
)(_edge_body)


def _mm_body(x_ref, wlp_ref, whp_ref, wi_ref, blp_ref, bhp_ref, bi_ref,
             hlp_ref, hhp_ref, hi_ref):
    x = x_ref[...]
    hlp_ref[...] = (jnp.dot(x, wlp_ref[...], preferred_element_type=jnp.float32)
                    + blp_ref[...])
    hhp_ref[...] = (jnp.dot(x, whp_ref[...], preferred_element_type=jnp.float32)
                    + bhp_ref[...])
    hi = jnp.dot(x, wi_ref[...], preferred_element_type=jnp.float32) + bi_ref[...]
    hi_ref[...] = jnp.maximum(hi, 0.0)


def _mm_stage(x, wlp, whp, wi, blp, bhp, bi):
    full = pl.BlockSpec((D, D), lambda i: (0, 0))
    bias = pl.BlockSpec((1, D), lambda i: (0, 0))
    row = pl.BlockSpec((BN, D), lambda i: (i, 0))
    return pl.pallas_call(
        _mm_body,
        grid=(N // BN,),
        in_specs=[row, full, full, full, bias, bias, bias],
        out_specs=[row, row, row],
        out_shape=[jax.ShapeDtypeStruct((N, D), jnp.float32)] * 3,
    )(x, wlp, whp, wi, blp, bhp, bi)


def _pre_body(hlp_ref, hhp_ref, degp_ref, hp_ref):
    d = degp_ref[...]
    deg = d[0, :, 0:1] + d[1, :, 0:1] + 1.0
    dinv = lax.rsqrt(deg)
    hp_ref[0, :, :] = dinv * hlp_ref[...]
    hp_ref[1, :, :] = dinv * hhp_ref[...]


def _pre_stage(hlp, hhp, degp):
    row = pl.BlockSpec((BN, D), lambda i: (i, 0))
    return pl.pallas_call(
        _pre_body,
        grid=(N // BN,),
        in_specs=[row, row, pl.BlockSpec((NC, BN, D), lambda i: (0, i, 0))],
        out_specs=pl.BlockSpec((NC, BN, D), lambda i: (0, i, 0)),
        out_shape=jax.ShapeDtypeStruct((NC, N, D), jnp.float32),
    )(hlp, hhp, degp)


def _epi_body(s_ref, hhp_ref, hi_ref, degp_ref, gw_ref, gb_ref, o_ref):
    d = degp_ref[...]
    deg = d[0, :, 0:1] + d[1, :, 0:1] + 1.0
    dinv = lax.rsqrt(deg)
    s = s_ref[...]
    h_lp = jnp.maximum(dinv * s[0], 0.0)
    h_hp = jnp.maximum(hhp_ref[...] - dinv * s[1], 0.0)
    h_i = hi_ref[...]
    gw = gw_ref[...]
    gb = gb_ref[...]

    def gate(h, k):
        z = jnp.sum(h * gw[k:k + 1, :], axis=1, keepdims=True) + gb[k:k + 1, 0:1]
        return 1.0 / (1.0 + jnp.exp(-z))

    out = gate(h_hp, 0) * h_hp + gate(h_lp, 1) * h_lp + gate(h_i, 2) * h_i
    m = jnp.max(out, axis=1, keepdims=True)
    lse = jnp.log(jnp.sum(jnp.exp(out - m), axis=1, keepdims=True)) + m
    o_ref[...] = out - lse


def _epi_stage(s3, hhp, hi, degp, gw, gb):
    row = pl.BlockSpec((BN, D), lambda i: (i, 0))
    return pl.pallas_call(
        _epi_body,
        grid=(N // BN,),
        in_specs=[pl.BlockSpec((NC, BN, D), lambda i: (0, i, 0)), row, row,
                  pl.BlockSpec((NC, BN, D), lambda i: (0, i, 0)),
                  pl.BlockSpec((3, D), lambda i: (0, 0)),
                  pl.BlockSpec((3, D), lambda i: (0, 0))],
        out_specs=row,
        out_shape=jax.ShapeDtypeStruct((N, D), jnp.float32),
    )(s3, hhp, hi, degp, gw, gb)


def kernel(x, edge_index, W_hp, b_hp, W_lp, b_lp, W_i, b_i, wh, bh, wl, bl, wi, bi):
    src = edge_index[0].astype(jnp.int32)
    dst = edge_index[1].astype(jnp.int32)
    zer = jnp.zeros((RPT, D), jnp.float32)
    one = jnp.ones((K, D), jnp.float32)

    hlp, hhp, hi = _mm_stage(
        x, W_lp, W_hp, W_i,
        b_lp.reshape(1, D), b_hp.reshape(1, D), b_i.reshape(1, D))
    degp = _deg_kernel(dst, zer, one).reshape(NC, N, D)
    hp = _pre_stage(hlp, hhp, degp)

    src2 = jnp.concatenate([src, src + N])
    s2 = _edge_kernel(hp.reshape(NC * N, D), src2, dst)

    gw = jnp.concatenate([wh, wl, wi], axis=1).T
    gb = jnp.broadcast_to(jnp.concatenate([bh, bl, bi])[:, None], (3, D))
    return _epi_stage(s2.reshape(NC, N, D), hhp, hi, degp, gw, gb)


# speedup vs baseline: 28.5737x; 1.1136x over previous
import functools

import jax
import jax.numpy as jnp
from jax import lax
from jax.experimental import pallas as pl
from jax.experimental.pallas import tpu as pltpu
from jax.experimental.pallas import tpu_sc as plsc

N = 10000
E = 320000
D = 128
NC = 2
NS = 16
K = 80
RPT = 624
TAIL = N - NS * RPT
EPW_A = E // (NC * NS)
EPT_C = E // NS
CH_A = EPW_A // K
CH_C = EPT_C // K
NB_A = 25
NB_C = 4
NG_C = CH_C // NB_C
BN = 1000


def _sc_mesh():
    return plsc.VectorSubcoreMesh(core_axis_name="c", subcore_axis_name="s")


def _m8(v):
    return pl.multiple_of(v, 8)


HW = 8


def _deg_body(dst1d, zh, out, hist, dbuf, sums, sem):
    c = lax.axis_index("c")
    s = lax.axis_index("s")
    w = s * NC + c
    cz = pltpu.async_copy(zh, hist, sem)
    ci = pltpu.async_copy(dst1d.at[pl.ds(_m8(w * EPW_A), EPW_A)], dbuf, sem)
    cz.wait()
    ci.wait()

    iota = lax.broadcasted_iota(jnp.int32, (16,), 0)
    col = lax.bitwise_and(iota, HW - 1)
    mlo = iota < HW
    mhi = iota >= HW
    ones16 = jnp.ones((16,), jnp.float32)

    def ed(i, carry):
        idx = dbuf[pl.ds(i * 16, 16)]
        addr = idx * HW + col
        plsc.addupdate_scatter(hist, [addr], ones16, mask=mlo)
        plsc.addupdate_scatter(hist, [addr], ones16, mask=mhi)
        return carry

    lax.fori_loop(0, EPW_A // 16, ed, 0)

    stride = iota * HW

    def cs(i, carry):
        base = i * (16 * HW)
        a = plsc.load_gather(hist, [base + stride])
        for cc in range(1, HW):
            a = a + plsc.load_gather(hist, [base + stride + cc])
        sums[pl.ds(i * 16, 16)] = a
        return carry

    lax.fori_loop(0, N // 16, cs, 0)
    pltpu.sync_copy(sums, out.at[w])


_deg_kernel = functools.partial(
    pl.kernel,
    out_type=jax.ShapeDtypeStruct((NS * NC, N), jnp.float32),
    mesh=_sc_mesh(),
    compiler_params=pltpu.CompilerParams(needs_layout_passes=False),
    scratch_types=[
        pltpu.VMEM((N * HW,), jnp.float32),
        pltpu.VMEM((EPW_A,), jnp.int32),
        pltpu.VMEM((N,), jnp.float32),
        pltpu.SemaphoreType.DMA,
    ],
)(_deg_body)


def _edge_body(hp, src1d, dst1d, out, acc, rows, sidx, didx, semi, semg, sems):
    c = lax.axis_index("c")
    s = lax.axis_index("s")
    off = c * N
    b0 = _m8(s * RPT)
    pltpu.sync_copy(hp.at[pl.ds(_m8(off + b0), RPT), :],
                    acc.at[pl.ds(b0, RPT), :])

    @pl.when(s == NS - 1)
    def _():
        pltpu.sync_copy(hp.at[pl.ds(_m8(off + NS * RPT), TAIL), :],
                        acc.at[pl.ds(_m8(NS * RPT), TAIL), :])

    plsc.subcore_barrier()

    def chunks(base, nb):
        lsrc = [pltpu.async_copy(
            src1d.at[pl.ds(_m8(c * E + (base + j) * K), K)], sidx.at[j], semi)
            for j in range(nb)]
        ldst = [pltpu.async_copy(
            dst1d.at[pl.ds(_m8((base + j) * K), K)], didx.at[j], semi)
            for j in range(nb)]
        gats = []
        for j in range(nb):
            lsrc[j].wait()
            gats.append(pltpu.async_copy(hp.at[sidx.at[j]], rows.at[j], semg))
        scats = []
        for j in range(nb):
            gats[j].wait()
            ldst[j].wait()
            scats.append(pltpu.async_copy(rows.at[j], acc.at[didx.at[j]],
                                          sems, add=True))
        for cp in scats:
            cp.wait()

    def group(g, carry):
        chunks(s * CH_C + g * NB_C, NB_C)
        return carry

    lax.fori_loop(0, NG_C, group, 0)
    chunks(s * CH_C + NG_C * NB_C, CH_C - NG_C * NB_C)
    plsc.subcore_barrier()
    pltpu.sync_copy(acc.at[pl.ds(b0, RPT), :],
                    out.at[pl.ds(_m8(off + b0), RPT), :])

    @pl.when(s == NS - 1)
    def _():
        pltpu.sync_copy(acc.at[pl.ds(_m8(NS * RPT), TAIL), :],
                        out.at[pl.ds(_m8(off + NS * RPT), TAIL), :])


_edge_kernel = functools.partial(
    pl.kernel,
    out_type=jax.ShapeDtypeStruct((NC * N, D), jnp.float32),
    mesh=_sc_mesh(),
    scratch_types=[
        pltpu.VMEM_SHARED((N, D), jnp.float32),
        pltpu.VMEM((NB_C, K, D), jnp.float32),
        pltpu.VMEM((NB_C, K), jnp.int32),
        pltpu.VMEM((NB_C, K), jnp.int32),
        pltpu.SemaphoreType.DMA,
        pltpu.SemaphoreType.DMA,
        pltpu.SemaphoreType.DMA,
    ],
)(_edge_body)


def _mm_body(x_ref, wlp_ref, whp_ref, wi_ref, blp_ref, bhp_ref, bi_ref,
             hlp_ref, hhp_ref, hi_ref):
    x = x_ref[...]
    hlp_ref[...] = (jnp.dot(x, wlp_ref[...], preferred_element_type=jnp.float32)
                    + blp_ref[...])
    hhp_ref[...] = (jnp.dot(x, whp_ref[...], preferred_element_type=jnp.float32)
                    + bhp_ref[...])
    hi = jnp.dot(x, wi_ref[...], preferred_element_type=jnp.float32) + bi_ref[...]
    hi_ref[...] = jnp.maximum(hi, 0.0)


def _mm_stage(x, wlp, whp, wi, blp, bhp, bi):
    full = pl.BlockSpec((D, D), lambda i: (0, 0))
    bias = pl.BlockSpec((1, D), lambda i: (0, 0))
    row = pl.BlockSpec((BN, D), lambda i: (i, 0))
    return pl.pallas_call(
        _mm_body,
        grid=(N // BN,),
        in_specs=[row, full, full, full, bias, bias, bias],
        out_specs=[row, row, row],
        out_shape=[jax.ShapeDtypeStruct((N, D), jnp.float32)] * 3,
    )(x, wlp, whp, wi, blp, bhp, bi)


def _dcol_body(degs_ref, o_ref):
    ones_w = jnp.ones((NS * NC, 1), jnp.float32)
    deg = lax.dot_general(degs_ref[...], ones_w, (((0,), (0,)), ((), ())),
                          preferred_element_type=jnp.float32) + 1.0
    o_ref[...] = lax.rsqrt(deg)


def _dcol_stage(degs):
    return pl.pallas_call(
        _dcol_body,
        grid=(1,),
        in_specs=[pl.BlockSpec((NS * NC, N), lambda i: (0, 0))],
        out_specs=pl.BlockSpec((N, 1), lambda i: (0, 0)),
        out_shape=jax.ShapeDtypeStruct((N, 1), jnp.float32),
    )(degs)


def _pre_body(hlp_ref, hhp_ref, dinv_ref, hp_ref):
    dinv = dinv_ref[...]
    hp_ref[0, :, :] = dinv * hlp_ref[...]
    hp_ref[1, :, :] = dinv * hhp_ref[...]


def _pre_stage(hlp, hhp, dinv):
    row = pl.BlockSpec((BN, D), lambda i: (i, 0))
    return pl.pallas_call(
        _pre_body,
        grid=(N // BN,),
        in_specs=[row, row, pl.BlockSpec((BN, 1), lambda i: (i, 0))],
        out_specs=pl.BlockSpec((NC, BN, D), lambda i: (0, i, 0)),
        out_shape=jax.ShapeDtypeStruct((NC, N, D), jnp.float32),
    )(hlp, hhp, dinv)


def _epi_body(s_ref, hhp_ref, hi_ref, dinv_ref, gw_ref, gb_ref, o_ref):
    dinv = dinv_ref[...]
    s = s_ref[...]
    h_lp = jnp.maximum(dinv * s[0], 0.0)
    h_hp = jnp.maximum(hhp_ref[...] - dinv * s[1], 0.0)
    h_i = hi_ref[...]
    gw = gw_ref[...]
    gb = gb_ref[...]

    def gate(h, k):
        z = jnp.sum(h * gw[k:k + 1, :], axis=1, keepdims=True) + gb[k:k + 1, 0:1]
        return 1.0 / (1.0 + jnp.exp(-z))

    out = gate(h_hp, 0) * h_hp + gate(h_lp, 1) * h_lp + gate(h_i, 2) * h_i
    m = jnp.max(out, axis=1, keepdims=True)
    lse = jnp.log(jnp.sum(jnp.exp(out - m), axis=1, keepdims=True)) + m
    o_ref[...] = out - lse


def _epi_stage(s3, hhp, hi, dinv, gw, gb):
    row = pl.BlockSpec((BN, D), lambda i: (i, 0))
    return pl.pallas_call(
        _epi_body,
        grid=(N // BN,),
        in_specs=[pl.BlockSpec((NC, BN, D), lambda i: (0, i, 0)), row, row,
                  pl.BlockSpec((BN, 1), lambda i: (i, 0)),
                  pl.BlockSpec((3, D), lambda i: (0, 0)),
                  pl.BlockSpec((3, D), lambda i: (0, 0))],
        out_specs=row,
        out_shape=jax.ShapeDtypeStruct((N, D), jnp.float32),
    )(s3, hhp, hi, dinv, gw, gb)


def kernel(x, edge_index, W_hp, b_hp, W_lp, b_lp, W_i, b_i, wh, bh, wl, bl, wi, bi):
    src = edge_index[0].astype(jnp.int32)
    dst = edge_index[1].astype(jnp.int32)
    zh = jnp.zeros((N * HW,), jnp.float32)

    hlp, hhp, hi = _mm_stage(
        x, W_lp, W_hp, W_i,
        b_lp.reshape(1, D), b_hp.reshape(1, D), b_i.reshape(1, D))
    degs = _deg_kernel(dst, zh)
    dinv = _dcol_stage(degs)
    hp = _pre_stage(hlp, hhp, dinv)

    src2 = jnp.concatenate([src, src + N])
    s2 = _edge_kernel(hp.reshape(NC * N, D), src2, dst)

    gw = jnp.concatenate([wh, wl, wi], axis=1).T
    gb = jnp.broadcast_to(jnp.concatenate([bh, bl, bi])[:, None], (3, D))
    return _epi_stage(s2.reshape(NC, N, D), hhp, hi, dinv, gw, gb)


# speedup vs baseline: 28.6158x; 1.0015x over previous
import functools

import jax
import jax.numpy as jnp
from jax import lax
from jax.experimental import pallas as pl
from jax.experimental.pallas import tpu as pltpu
from jax.experimental.pallas import tpu_sc as plsc

N = 10000
E = 320000
D = 128
NC = 2
NS = 16
K = 80
RPT = 624
TAIL = N - NS * RPT
EPW_A = E // (NC * NS)
EPT_C = E // NS
CH_A = EPW_A // K
CH_C = EPT_C // K
NB_A = 25
NB_C = 4
NG_C = CH_C // NB_C
NP = 10240
BN = 1024


def _sc_mesh():
    return plsc.VectorSubcoreMesh(core_axis_name="c", subcore_axis_name="s")


def _m8(v):
    return pl.multiple_of(v, 8)


HW = 8


def _deg_body(dst1d, zh, out, hist, dbuf, sums, sem):
    c = lax.axis_index("c")
    s = lax.axis_index("s")
    w = s * NC + c
    cz = pltpu.async_copy(zh, hist, sem)
    ci = pltpu.async_copy(dst1d.at[pl.ds(_m8(w * EPW_A), EPW_A)], dbuf, sem)
    cz.wait()
    ci.wait()

    iota = lax.broadcasted_iota(jnp.int32, (16,), 0)
    col = lax.bitwise_and(iota, HW - 1)
    mlo = iota < HW
    mhi = iota >= HW
    ones16 = jnp.ones((16,), jnp.float32)

    def ed(i, carry):
        idx = dbuf[pl.ds(i * 16, 16)]
        addr = idx * HW + col
        plsc.addupdate_scatter(hist, [addr], ones16, mask=mlo)
        plsc.addupdate_scatter(hist, [addr], ones16, mask=mhi)
        return carry

    lax.fori_loop(0, EPW_A // 16, ed, 0)

    stride = iota * HW

    def cs(i, carry):
        base = i * (16 * HW)
        a = plsc.load_gather(hist, [base + stride])
        for cc in range(1, HW):
            a = a + plsc.load_gather(hist, [base + stride + cc])
        sums[pl.ds(i * 16, 16)] = a
        return carry

    lax.fori_loop(0, N // 16, cs, 0)
    zero16 = jnp.zeros((16,), jnp.float32)
    for t in range(N // 16, NP // 16):
        sums[pl.ds(t * 16, 16)] = zero16
    pltpu.sync_copy(sums, out.at[w])


_deg_kernel = functools.partial(
    pl.kernel,
    out_type=jax.ShapeDtypeStruct((NS * NC, NP), jnp.float32),
    mesh=_sc_mesh(),
    compiler_params=pltpu.CompilerParams(needs_layout_passes=False),
    scratch_types=[
        pltpu.VMEM((N * HW,), jnp.float32),
        pltpu.VMEM((EPW_A,), jnp.int32),
        pltpu.VMEM((NP,), jnp.float32),
        pltpu.SemaphoreType.DMA,
    ],
)(_deg_body)


def _edge_body(hp, src1d, dst1d, out, acc, rows, sidx, didx, semi, semg, sems):
    c = lax.axis_index("c")
    s = lax.axis_index("s")
    off = c * NP
    b0 = _m8(s * RPT)
    pltpu.sync_copy(hp.at[pl.ds(_m8(off + b0), RPT), :],
                    acc.at[pl.ds(b0, RPT), :])

    @pl.when(s == NS - 1)
    def _():
        pltpu.sync_copy(hp.at[pl.ds(_m8(off + NS * RPT), TAIL), :],
                        acc.at[pl.ds(_m8(NS * RPT), TAIL), :])

    plsc.subcore_barrier()

    def chunks(base, nb):
        lsrc = [pltpu.async_copy(
            src1d.at[pl.ds(_m8(c * E + (base + j) * K), K)], sidx.at[j], semi)
            for j in range(nb)]
        ldst = [pltpu.async_copy(
            dst1d.at[pl.ds(_m8((base + j) * K), K)], didx.at[j], semi)
            for j in range(nb)]
        gats = []
        for j in range(nb):
            lsrc[j].wait()
            gats.append(pltpu.async_copy(hp.at[sidx.at[j]], rows.at[j], semg))
        scats = []
        for j in range(nb):
            gats[j].wait()
            ldst[j].wait()
            scats.append(pltpu.async_copy(rows.at[j], acc.at[didx.at[j]],
                                          sems, add=True))
        for cp in scats:
            cp.wait()

    def group(g, carry):
        chunks(s * CH_C + g * NB_C, NB_C)
        return carry

    lax.fori_loop(0, NG_C, group, 0)
    chunks(s * CH_C + NG_C * NB_C, CH_C - NG_C * NB_C)
    plsc.subcore_barrier()
    pltpu.sync_copy(acc.at[pl.ds(b0, RPT), :],
                    out.at[pl.ds(_m8(off + b0), RPT), :])

    @pl.when(s == NS - 1)
    def _():
        pltpu.sync_copy(acc.at[pl.ds(_m8(NS * RPT), TAIL), :],
                        out.at[pl.ds(_m8(off + NS * RPT), TAIL), :])


_edge_kernel = functools.partial(
    pl.kernel,
    out_type=jax.ShapeDtypeStruct((NC * NP, D), jnp.float32),
    mesh=_sc_mesh(),
    scratch_types=[
        pltpu.VMEM_SHARED((N, D), jnp.float32),
        pltpu.VMEM((NB_C, K, D), jnp.float32),
        pltpu.VMEM((NB_C, K), jnp.int32),
        pltpu.VMEM((NB_C, K), jnp.int32),
        pltpu.SemaphoreType.DMA,
        pltpu.SemaphoreType.DMA,
        pltpu.SemaphoreType.DMA,
    ],
)(_edge_body)


def _dinv_blk(degs_ref):
    ones_w = jnp.ones((NS * NC, 1), jnp.float32)
    deg = lax.dot_general(degs_ref[...], ones_w, (((0,), (0,)), ((), ())),
                          preferred_element_type=jnp.float32) + 1.0
    return lax.rsqrt(deg)


def _dense_body(x_ref, wlp_ref, whp_ref, wi_ref, blp_ref, bhp_ref, bi_ref,
                degs_ref, hp_ref, hhp_ref, hi_ref):
    dinv = _dinv_blk(degs_ref)
    x = x_ref[...]
    hlp = jnp.dot(x, wlp_ref[...], preferred_element_type=jnp.float32) + blp_ref[...]
    hhp = jnp.dot(x, whp_ref[...], preferred_element_type=jnp.float32) + bhp_ref[...]
    hi = jnp.dot(x, wi_ref[...], preferred_element_type=jnp.float32) + bi_ref[...]
    hp_ref[0, :, :] = dinv * hlp
    hp_ref[1, :, :] = dinv * hhp
    hhp_ref[...] = hhp
    hi_ref[...] = jnp.maximum(hi, 0.0)


def _dense_stage(x, wlp, whp, wi, blp, bhp, bi, degs):
    full = pl.BlockSpec((D, D), lambda i: (0, 0))
    bias = pl.BlockSpec((1, D), lambda i: (0, 0))
    row = pl.BlockSpec((BN, D), lambda i: (i, 0))
    return pl.pallas_call(
        _dense_body,
        grid=(NP // BN,),
        in_specs=[row, full, full, full, bias, bias, bias,
                  pl.BlockSpec((NS * NC, BN), lambda i: (0, i))],
        out_specs=[pl.BlockSpec((NC, BN, D), lambda i: (0, i, 0)), row, row],
        out_shape=[
            jax.ShapeDtypeStruct((NC, NP, D), jnp.float32),
            jax.ShapeDtypeStruct((NP, D), jnp.float32),
            jax.ShapeDtypeStruct((NP, D), jnp.float32),
        ],
    )(x, wlp, whp, wi, blp, bhp, bi, degs)


def _epi_body(s_ref, hhp_ref, hi_ref, degs_ref, gw_ref, gb_ref, o_ref):
    dinv = _dinv_blk(degs_ref)
    s = s_ref[...]
    h_lp = jnp.maximum(dinv * s[0], 0.0)
    h_hp = jnp.maximum(hhp_ref[...] - dinv * s[1], 0.0)
    h_i = hi_ref[...]
    gw = gw_ref[...]
    gb = gb_ref[...]

    def gate(h, k):
        z = jnp.sum(h * gw[k:k + 1, :], axis=1, keepdims=True) + gb[k:k + 1, 0:1]
        return 1.0 / (1.0 + jnp.exp(-z))

    out = gate(h_hp, 0) * h_hp + gate(h_lp, 1) * h_lp + gate(h_i, 2) * h_i
    m = jnp.max(out, axis=1, keepdims=True)
    lse = jnp.log(jnp.sum(jnp.exp(out - m), axis=1, keepdims=True)) + m
    o_ref[...] = out - lse


def _epi_stage(s3, hhp, hi, degs, gw, gb):
    row = pl.BlockSpec((BN, D), lambda i: (i, 0))
    return pl.pallas_call(
        _epi_body,
        grid=(NP // BN,),
        in_specs=[pl.BlockSpec((NC, BN, D), lambda i: (0, i, 0)), row, row,
                  pl.BlockSpec((NS * NC, BN), lambda i: (0, i)),
                  pl.BlockSpec((3, D), lambda i: (0, 0)),
                  pl.BlockSpec((3, D), lambda i: (0, 0))],
        out_specs=row,
        out_shape=jax.ShapeDtypeStruct((NP, D), jnp.float32),
    )(s3, hhp, hi, degs, gw, gb)


def kernel(x, edge_index, W_hp, b_hp, W_lp, b_lp, W_i, b_i, wh, bh, wl, bl, wi, bi):
    src = edge_index[0].astype(jnp.int32)
    dst = edge_index[1].astype(jnp.int32)
    zh = jnp.zeros((N * HW,), jnp.float32)
    xp = jnp.pad(x, ((0, NP - N), (0, 0)))

    degs = _deg_kernel(dst, zh)
    hp, hhp, hi = _dense_stage(
        xp, W_lp, W_hp, W_i,
        b_lp.reshape(1, D), b_hp.reshape(1, D), b_i.reshape(1, D), degs)

    src2 = jnp.concatenate([src, src + NP])
    s2 = _edge_kernel(hp.reshape(NC * NP, D), src2, dst)

    gw = jnp.concatenate([wh, wl, wi], axis=1).T
    gb = jnp.broadcast_to(jnp.concatenate([bh, bl, bi])[:, None], (3, D))
    return _epi_stage(s2.reshape(NC, NP, D), hhp, hi, degs, gw, gb)[:N]
